# baseline (speedup 1.0000x reference)
import functools

import jax
import jax.numpy as jnp
from jax import lax
from jax.experimental import pallas as pl
from jax.experimental.pallas import tpu as pltpu
from jax.experimental.pallas import tpu_sc as plsc

NUM_EMBEDDINGS = 1000000
FEATURES = 32
BATCH = 16384
HIST = 50

_B = BATCH * HIST

_info = plsc.get_sparse_core_info()
_NC, _NS = _info.num_cores, _info.num_subcores
_NW = _NC * _NS
_LANES = 128
_NBLK = NUM_EMBEDDINGS // _LANES
_TAIL = NUM_EMBEDDINGS - _NBLK * _LANES
_KFULL = _NBLK // _NW
_BB_PER_W = (BATCH // _LANES) // _NW
_BBC = _LANES * HIST


def _make_transpose_table():
    mesh = plsc.VectorSubcoreMesh(core_axis_name="c", subcore_axis_name="s")

    @functools.partial(
        pl.kernel,
        mesh=mesh,
        compiler_params=pltpu.CompilerParams(use_tc_tiling_on_sc=True, needs_layout_passes=False),
        out_type=jax.ShapeDtypeStruct(((_NBLK * _LANES + _LANES) * FEATURES,),
                                      jnp.float32),
        scratch_types=(
            [pltpu.VMEM((FEATURES, _LANES), jnp.float32) for _ in range(2)]
            + [pltpu.VMEM((_LANES * FEATURES,), jnp.float32) for _ in range(2)]
            + [pltpu.SemaphoreType.DMA for _ in range(4)]
        ),
    )
    def p(tt_hbm, lin_hbm, inb0, inb1, outb0, outb1, is0, is1, os0, os1):
        inb, outb = (inb0, inb1), (outb0, outb1)
        i_sem, o_sem = (is0, is1), (os0, os1)
        wid = lax.axis_index("s") * _NC + lax.axis_index("c")
        i32 = lax.iota(jnp.int32, 16) * FEATURES

        def start_in(k, b):
            pltpu.async_copy(
                tt_hbm.at[:, pl.ds((wid + _NW * k) * _LANES, _LANES)],
                inb[b], i_sem[b])

        def start_out(k, b):
            pltpu.async_copy(
                outb[b],
                lin_hbm.at[pl.ds((wid + _NW * k) * _LANES * FEATURES,
                                 _LANES * FEATURES)],
                o_sem[b])

        def wait_in(b):
            pltpu.make_async_copy(
                tt_hbm.at[:, pl.ds(0, _LANES)], inb[b], i_sem[b]).wait()

        def wait_out(b):
            pltpu.make_async_copy(
                outb[b], lin_hbm.at[pl.ds(0, _LANES * FEATURES)],
                o_sem[b]).wait()

        def transpose_block(b, ncols):
            for j in range(FEATURES):
                for c8 in range(ncols // 16):
                    v = inb[b][j, pl.ds(c8 * 16, 16)]
                    plsc.store_scatter(outb[b], [i32 + (c8 * 16 * FEATURES + j)], v)

        start_in(0, 0)
        start_in(1, 1)

        @pl.loop(0, _KFULL, step=2)
        def body(k):
            for b in range(2):
                kk = k + b
                wait_in(b)

                @pl.when(kk >= 2)
                def _():
                    wait_out(b)

                transpose_block(b, _LANES)
                start_out(kk, b)

                @pl.when(kk + 2 < _KFULL)
                def _():
                    start_in(kk + 2, b)

        wait_out(0)
        wait_out(1)

        @pl.when(wid < 4)
        def _():
            pltpu.sync_copy(
                tt_hbm.at[:, pl.ds((wid + _NW * _KFULL) * _LANES, _LANES)],
                inb[0])
            transpose_block(0, _LANES)
            pltpu.sync_copy(
                outb[0],
                lin_hbm.at[pl.ds((wid + _NW * _KFULL) * _LANES * FEATURES,
                                 _LANES * FEATURES)])

    return p


def _make_gather():
    mesh = plsc.VectorSubcoreMesh(core_axis_name="c", subcore_axis_name="s")

    @functools.partial(
        pl.kernel,
        mesh=mesh,
        compiler_params=pltpu.CompilerParams(use_tc_tiling_on_sc=False, needs_layout_passes=False),
        out_type=jax.ShapeDtypeStruct((HIST, 4, BATCH // _LANES, 8, _LANES),
                                      jnp.float32),
        scratch_types=(
            [pltpu.VMEM((_BBC,), jnp.int32) for _ in range(3)]
            + [pltpu.VMEM((_LANES, FEATURES), jnp.float32) for _ in range(2)]
            + [pltpu.VMEM((FEATURES, _LANES), jnp.float32) for _ in range(2)]
            + [pltpu.SemaphoreType.DMA for _ in range(4)]
        ),
    )
    def g(idx_hbm, table_hbm, perm_hbm, out_hbm,
          perm_v, idxr, idxp, hr0, hr1, ht0, ht1, gs0, gs1, ws0, ws1):
        hrows, ht = (hr0, hr1), (ht0, ht1)
        g_sem, w_sem = (gs0, gs1), (ws0, ws1)
        wid = lax.axis_index("s") * _NC + lax.axis_index("c")
        iota = lax.iota(jnp.int32, 16)
        r0 = iota
        r1 = iota + 16

        pltpu.sync_copy(perm_hbm, perm_v)

        def start_gather(h, b):
            pltpu.async_copy(
                table_hbm.at[idxp.at[pl.ds(h * _LANES, _LANES)]],
                hrows[b], g_sem[b])

        def wait_gather(b):
            pltpu.make_async_copy(
                table_hbm.at[pl.ds(0, _LANES)], hrows[b], g_sem[b]).wait()

        def wait_writes(b):
            for _ in range(4):
                pltpu.make_async_copy(
                    ht[b].at[pl.ds(0, 8), :], out_hbm.at[0, 0, 0],
                    w_sem[b]).wait()

        def transpose_unit(b):
            for c in range(_LANES):
                col = iota * 0 + c
                v0 = hrows[b][c, pl.ds(0, 16)]
                plsc.store_scatter(ht[b], [r0, col], v0)
                v1 = hrows[b][c, pl.ds(16, 16)]
                plsc.store_scatter(ht[b], [r1, col], v1)

        @pl.loop(0, _BB_PER_W)
        def per_bb(s):
            bb = wid * _BB_PER_W + s
            pltpu.sync_copy(idx_hbm.at[pl.ds(bb * _BBC, _BBC)], idxr)

            @pl.loop(0, _BBC // 16)
            def permute(u):
                pv = perm_v[pl.ds(u * 16, 16)]
                idxp[pl.ds(u * 16, 16)] = plsc.load_gather(idxr, [pv])

            start_gather(0, 0)
            start_gather(1, 1)

            @pl.loop(0, HIST, step=2)
            def per_h(hh):
                for b in range(2):
                    h = hh + b
                    wait_gather(b)

                    @pl.when(h >= 2)
                    def _():
                        wait_writes(b)

                    transpose_unit(b)
                    for jb in range(4):
                        pltpu.async_copy(
                            ht[b].at[pl.ds(8 * jb, 8), :],
                            out_hbm.at[h, jb, bb], w_sem[b])

                    @pl.when(h + 2 < HIST)
                    def _():
                        start_gather(h + 2, b)

            wait_writes(0)
            wait_writes(1)

    return g


_transpose_table = _make_transpose_table()
_gather = _make_gather()


def kernel(inputs, embedding):
    lin = _transpose_table(embedding.T)
    tail = embedding[_NBLK * _LANES:, :].reshape(-1)
    lin = lax.dynamic_update_slice(lin, tail, (_NBLK * _LANES * FEATURES,))
    table2d = lin.reshape(_NBLK * _LANES + _LANES, FEATURES)
    d = jnp.arange(_BBC, dtype=jnp.int32)
    perm = (d % _LANES) * HIST + d // _LANES
    out5 = _gather(inputs.reshape(-1), table2d, perm)
    return out5.transpose(2, 4, 0, 1, 3).reshape(BATCH, HIST, FEATURES)


# speedup vs baseline: 1.2395x; 1.2395x over previous
import functools

import jax
import jax.numpy as jnp
from jax import lax
from jax.experimental import pallas as pl
from jax.experimental.pallas import tpu as pltpu
from jax.experimental.pallas import tpu_sc as plsc

NUM_EMBEDDINGS = 1000000
FEATURES = 32
BATCH = 16384
HIST = 50

_B = BATCH * HIST

_info = plsc.get_sparse_core_info()
_NC, _NS = _info.num_cores, _info.num_subcores
_NW = _NC * _NS
_LANES = 128
_NBLK = NUM_EMBEDDINGS // _LANES
_TAIL = NUM_EMBEDDINGS - _NBLK * _LANES
_KFULL = _NBLK // _NW
_BB_PER_W = (BATCH // _LANES) // _NW
_BBC = _LANES * HIST


def _make_transpose_table():
    mesh = plsc.VectorSubcoreMesh(core_axis_name="c", subcore_axis_name="s")

    @functools.partial(
        pl.kernel,
        mesh=mesh,
        compiler_params=pltpu.CompilerParams(use_tc_tiling_on_sc=True, needs_layout_passes=False),
        out_type=jax.ShapeDtypeStruct(((_NBLK * _LANES + _LANES) * FEATURES,),
                                      jnp.float32),
        scratch_types=(
            [pltpu.VMEM((FEATURES, _LANES), jnp.float32) for _ in range(2)]
            + [pltpu.VMEM((_LANES * FEATURES,), jnp.float32) for _ in range(2)]
            + [pltpu.SemaphoreType.DMA for _ in range(4)]
        ),
    )
    def p(tt_hbm, lin_hbm, inb0, inb1, outb0, outb1, is0, is1, os0, os1):
        inb, outb = (inb0, inb1), (outb0, outb1)
        i_sem, o_sem = (is0, is1), (os0, os1)
        wid = lax.axis_index("s") * _NC + lax.axis_index("c")
        iota = lax.iota(jnp.int32, 16)
        rot = [lax.rem(iota + d, 16) for d in range(16)]
        sv = [rot[d] * FEATURES + iota for d in range(16)]

        def start_in(k, b):
            pltpu.async_copy(
                tt_hbm.at[:, pl.ds((wid + _NW * k) * _LANES, _LANES)],
                inb[b], i_sem[b])

        def start_out(k, b):
            pltpu.async_copy(
                outb[b],
                lin_hbm.at[pl.ds((wid + _NW * k) * _LANES * FEATURES,
                                 _LANES * FEATURES)],
                o_sem[b])

        def wait_in(b):
            pltpu.make_async_copy(
                tt_hbm.at[:, pl.ds(0, _LANES)], inb[b], i_sem[b]).wait()

        def wait_out(b):
            pltpu.make_async_copy(
                outb[b], lin_hbm.at[pl.ds(0, _LANES * FEATURES)],
                o_sem[b]).wait()

        def transpose_block(b, ncols):
            for j0 in (0, 16):
                rowv = iota + j0
                for c0 in range(0, ncols, 16):
                    base = c0 * FEATURES + j0
                    for d in range(16):
                        colv = rot[d] + c0
                        v = plsc.load_gather(inb[b], [rowv, colv])
                        plsc.store_scatter(outb[b], [sv[d] + base], v)

        start_in(0, 0)
        start_in(1, 1)

        @pl.loop(0, _KFULL, step=2)
        def body(k):
            for b in range(2):
                kk = k + b
                wait_in(b)

                @pl.when(kk >= 2)
                def _():
                    wait_out(b)

                transpose_block(b, _LANES)
                start_out(kk, b)

                @pl.when(kk + 2 < _KFULL)
                def _():
                    start_in(kk + 2, b)

        wait_out(0)
        wait_out(1)

        @pl.when(wid < 4)
        def _():
            pltpu.sync_copy(
                tt_hbm.at[:, pl.ds((wid + _NW * _KFULL) * _LANES, _LANES)],
                inb[0])
            transpose_block(0, _LANES)
            pltpu.sync_copy(
                outb[0],
                lin_hbm.at[pl.ds((wid + _NW * _KFULL) * _LANES * FEATURES,
                                 _LANES * FEATURES)])

    return p


def _make_gather():
    mesh = plsc.VectorSubcoreMesh(core_axis_name="c", subcore_axis_name="s")

    @functools.partial(
        pl.kernel,
        mesh=mesh,
        compiler_params=pltpu.CompilerParams(use_tc_tiling_on_sc=False, needs_layout_passes=False),
        out_type=jax.ShapeDtypeStruct((HIST, 4, BATCH // _LANES, 8, _LANES),
                                      jnp.float32),
        scratch_types=(
            [pltpu.VMEM((_BBC,), jnp.int32) for _ in range(3)]
            + [pltpu.VMEM((_LANES, FEATURES), jnp.float32) for _ in range(2)]
            + [pltpu.VMEM((FEATURES, _LANES), jnp.float32) for _ in range(2)]
            + [pltpu.SemaphoreType.DMA for _ in range(4)]
        ),
    )
    def g(idx_hbm, table_hbm, perm_hbm, out_hbm,
          perm_v, idxr, idxp, hr0, hr1, ht0, ht1, gs0, gs1, ws0, ws1):
        hrows, ht = (hr0, hr1), (ht0, ht1)
        g_sem, w_sem = (gs0, gs1), (ws0, ws1)
        wid = lax.axis_index("s") * _NC + lax.axis_index("c")
        iota = lax.iota(jnp.int32, 16)
        rot = [lax.rem(iota + d, 16) for d in range(16)]

        pltpu.sync_copy(perm_hbm, perm_v)

        def start_gather(h, b):
            pltpu.async_copy(
                table_hbm.at[idxp.at[pl.ds(h * _LANES, _LANES)]],
                hrows[b], g_sem[b])

        def wait_gather(b):
            pltpu.make_async_copy(
                table_hbm.at[pl.ds(0, _LANES)], hrows[b], g_sem[b]).wait()

        def wait_writes(b):
            for _ in range(4):
                pltpu.make_async_copy(
                    ht[b].at[pl.ds(0, 8), :], out_hbm.at[0, 0, 0],
                    w_sem[b]).wait()

        def transpose_unit(b):
            for b0 in range(0, _LANES, 16):
                rowv = iota + b0
                for j0 in (0, 16):
                    for d in range(16):
                        colv = rot[d] + j0
                        v = plsc.load_gather(hrows[b], [rowv, colv])
                        plsc.store_scatter(ht[b], [colv, rowv], v)

        @pl.loop(0, _BB_PER_W)
        def per_bb(s):
            bb = wid * _BB_PER_W + s
            pltpu.sync_copy(idx_hbm.at[pl.ds(bb * _BBC, _BBC)], idxr)

            @pl.loop(0, _BBC // 16)
            def permute(u):
                pv = perm_v[pl.ds(u * 16, 16)]
                idxp[pl.ds(u * 16, 16)] = plsc.load_gather(idxr, [pv])

            start_gather(0, 0)
            start_gather(1, 1)

            @pl.loop(0, HIST, step=2)
            def per_h(hh):
                for b in range(2):
                    h = hh + b
                    wait_gather(b)

                    @pl.when(h >= 2)
                    def _():
                        wait_writes(b)

                    transpose_unit(b)
                    for jb in range(4):
                        pltpu.async_copy(
                            ht[b].at[pl.ds(8 * jb, 8), :],
                            out_hbm.at[h, jb, bb], w_sem[b])

                    @pl.when(h + 2 < HIST)
                    def _():
                        start_gather(h + 2, b)

            wait_writes(0)
            wait_writes(1)

    return g


_transpose_table = _make_transpose_table()
_gather = _make_gather()


def kernel(inputs, embedding):
    lin = _transpose_table(embedding.T)
    tail = embedding[_NBLK * _LANES:, :].reshape(-1)
    lin = lax.dynamic_update_slice(lin, tail, (_NBLK * _LANES * FEATURES,))
    table2d = lin.reshape(_NBLK * _LANES + _LANES, FEATURES)
    d = jnp.arange(_BBC, dtype=jnp.int32)
    perm = (d % _LANES) * HIST + d // _LANES
    out5 = _gather(inputs.reshape(-1), table2d, perm)
    return out5.transpose(2, 4, 0, 1, 3).reshape(BATCH, HIST, FEATURES)
